# baseline (speedup 1.0000x reference)
import functools

import jax
import jax.numpy as jnp
from jax import lax
from jax.experimental import pallas as pl
from jax.experimental.pallas import tpu as pltpu
from jax.experimental.pallas import tpu_sc as plsc

N = 10000
C = 128
K = 16
KD = 32
E = N * K
NEG = -3.0e38
THRESH = -1.0e38

NC = 2
NS = 16
NW = NC * NS
R = 320
NPAD = NW * R
DUMMY = R

CHUNK = 4000
NCH = E // CHUNK
GPC = CHUNK // 16
BATCH = 128
RCAP = 512
RING = 16 * RCAP
CLCAP = RING + 16
EPB = BATCH // 16
PRESS = RCAP - 126


def _lrelu(x):
    return jnp.where(x >= 0, x, 0.2 * x)


def _mm_kernel(x_ref, w_ref, b_ref, o_ref):
    o_ref[...] = jnp.dot(x_ref[...], w_ref[...],
                         preferred_element_type=jnp.float32) + b_ref[...]


def _mm(x, w, b):
    n, cin = x.shape
    cout = w.shape[1]
    blk = 400
    return pl.pallas_call(
        _mm_kernel,
        grid=(n // blk,),
        in_specs=[
            pl.BlockSpec((blk, cin), lambda i: (i, 0)),
            pl.BlockSpec((cin, cout), lambda i: (0, 0)),
            pl.BlockSpec((cout,), lambda i: (0,)),
        ],
        out_specs=pl.BlockSpec((blk, cout), lambda i: (i, 0)),
        out_shape=jax.ShapeDtypeStruct((n, cout), jnp.float32),
    )(x, w, b)


def _pq_kernel(x_ref, w_ref, b_ref, p_ref, q_ref):
    pq = jnp.dot(x_ref[...], w_ref[...],
                 preferred_element_type=jnp.float32) + b_ref[...]
    p_ref[...] = pq[:, :C]
    q_ref[...] = pq[:, C:].astype(jnp.bfloat16)


def _pq_mm(x, w, b):
    n, cin = x.shape
    blk = 400
    return pl.pallas_call(
        _pq_kernel,
        grid=(n // blk,),
        in_specs=[
            pl.BlockSpec((blk, cin), lambda i: (i, 0)),
            pl.BlockSpec((cin, 2 * C), lambda i: (0, 0)),
            pl.BlockSpec((2 * C,), lambda i: (0,)),
        ],
        out_specs=[
            pl.BlockSpec((blk, C), lambda i: (i, 0)),
            pl.BlockSpec((blk, C), lambda i: (i, 0)),
        ],
        out_shape=[
            jax.ShapeDtypeStruct((n, C), jnp.float32),
            jax.ShapeDtypeStruct((n, C), jnp.bfloat16),
        ],
    )(x, w, b)


def _h_mm_kernel(p_ref, s_ref, x_ref, w_ref, b_ref, h_ref, p1_ref, q1_ref):
    s = s_ref[...].astype(jnp.float32)
    h = jnp.where(s > THRESH, _lrelu(p_ref[...] + s), 0.0)
    h_ref[...] = h
    f = jnp.concatenate([x_ref[...], h], axis=-1)
    pq = jnp.dot(f, w_ref[...],
                 preferred_element_type=jnp.float32) + b_ref[...]
    p1_ref[...] = pq[:, :C]
    q1_ref[...] = pq[:, C:].astype(jnp.bfloat16)


def _h_and_mm(p, s, x, w, b):
    n = x.shape[0]
    cin = x.shape[1] + C
    blk = 400
    return pl.pallas_call(
        _h_mm_kernel,
        grid=(n // blk,),
        in_specs=[
            pl.BlockSpec((blk, C), lambda i: (i, 0)),
            pl.BlockSpec((blk, C), lambda i: (i, 0)),
            pl.BlockSpec((blk, x.shape[1]), lambda i: (i, 0)),
            pl.BlockSpec((cin, 2 * C), lambda i: (0, 0)),
            pl.BlockSpec((2 * C,), lambda i: (0,)),
        ],
        out_specs=[
            pl.BlockSpec((blk, C), lambda i: (i, 0)),
            pl.BlockSpec((blk, C), lambda i: (i, 0)),
            pl.BlockSpec((blk, C), lambda i: (i, 0)),
        ],
        out_shape=[
            jax.ShapeDtypeStruct((n, C), jnp.float32),
            jax.ShapeDtypeStruct((n, C), jnp.float32),
            jax.ShapeDtypeStruct((n, C), jnp.bfloat16),
        ],
    )(p, s, x, w, b)


def _final_kernel(x_ref, h00_ref, p10_ref, s10_ref, h01_ref, p11_ref,
                  s11_ref, wf0_ref, bf0_ref, wf1_ref, bf1_ref, o_ref):
    x = x_ref[...]
    outs = []
    for h0, p1, s1, wf, bf in (
        (h00_ref, p10_ref, s10_ref, wf0_ref, bf0_ref),
        (h01_ref, p11_ref, s11_ref, wf1_ref, bf1_ref),
    ):
        s = s1[...].astype(jnp.float32)
        h1 = jnp.where(s > THRESH, _lrelu(p1[...] + s), 0.0)
        f = jnp.concatenate([x, h0[...], h1], axis=-1)
        outs.append(jnp.dot(f, wf[...],
                            preferred_element_type=jnp.float32) + bf[...])
    o_ref[...] = jnp.maximum(outs[0], outs[1]) + x


def _final(x, h00, p10, s10, h01, p11, s11, wf0, bf0, wf1, bf1):
    n = x.shape[0]
    blk = 400
    row = lambda i: (i, 0)
    full = lambda i: (0, 0)
    return pl.pallas_call(
        _final_kernel,
        grid=(n // blk,),
        in_specs=[
            pl.BlockSpec((blk, C), row), pl.BlockSpec((blk, C), row),
            pl.BlockSpec((blk, C), row), pl.BlockSpec((blk, C), row),
            pl.BlockSpec((blk, C), row), pl.BlockSpec((blk, C), row),
            pl.BlockSpec((blk, C), row),
            pl.BlockSpec((3 * C, C), full), pl.BlockSpec((C,), lambda i: (0,)),
            pl.BlockSpec((3 * C, C), full), pl.BlockSpec((C,), lambda i: (0,)),
        ],
        out_specs=pl.BlockSpec((blk, C), row),
        out_shape=jax.ShapeDtypeStruct((n, C), jnp.float32),
    )(x, h00, p10, s10, h01, p11, s11, wf0, bf0, wf1, bf1)


def _segmax_body(src_hbm, dst_hbm, q_hbm, out_hbm,
                 dstb0, dstb1, srcb0, srcb1, cl, gidx, lst, rowb, acc,
                 sem_scan, sem_g):
    dstb = (dstb0, dstb1)
    srcb = (srcb0, srcb1)
    wid = lax.axis_index("s") * NC + lax.axis_index("c")
    lo = wid * R
    hi = lo + R
    negv = jnp.full((32,), NEG, jnp.bfloat16)

    def init_body(r, carry):
        for v in range(4):
            acc[r, pl.ds(v * 32, 32)] = negv
        return carry
    lax.fori_loop(0, R + 1, init_body, 0)

    def start_chunk(t, par):
        pltpu.make_async_copy(
            dst_hbm.at[pl.ds(t * CHUNK, CHUNK)], dstb[par], sem_scan).start()
        pltpu.make_async_copy(
            src_hbm.at[pl.ds(t * CHUNK, CHUNK)], srcb[par], sem_scan).start()

    def wait_chunk(par):
        pltpu.make_async_copy(
            dst_hbm.at[pl.ds(0, CHUNK)], dstb[par], sem_scan).wait()
        pltpu.make_async_copy(
            src_hbm.at[pl.ds(0, CHUNK)], srcb[par], sem_scan).wait()

    lanes = lax.iota(jnp.int32, 16)

    def start_gather(nf):
        par = (nf % 2) * BATCH
        fb = pl.multiple_of((nf * BATCH) % RING, BATCH)
        for i in range(EPB):
            pv = cl[pl.ds(fb + i * 16, 16)]
            gidx[pl.ds(par + i * 16, 16)] = pv >> 9
            lst[pl.ds(par + i * 16, 16)] = pv & 511
        pltpu.make_async_copy(
            q_hbm.at[gidx.at[pl.ds(par, BATCH)]],
            rowb.at[pl.ds(par, BATCH)], sem_g).start()

    def rmw_batch(nf):
        par = (nf % 2) * BATCH
        pltpu.make_async_copy(
            q_hbm.at[gidx.at[pl.ds(par, BATCH)]],
            rowb.at[pl.ds(par, BATCH)], sem_g).wait()

        def rmw_body(i, carry):
            lv = lst[pl.ds(par + i * 16, 16)]
            for j in range(16):
                ld = lv[j]
                r = par + i * 16 + j
                g = [plsc.bitcast(rowb[r, pl.ds(v * 16, 16)], jnp.bfloat16)
                     for v in range(4)]
                a = [acc[ld, pl.ds(v * 32, 32)] for v in range(4)]
                for v in range(4):
                    acc[ld, pl.ds(v * 32, 32)] = jnp.maximum(a[v], g[v])
            return carry
        lax.fori_loop(0, 8, rmw_body, 0)

    def scan_group(base, cposv, dbuf, sbuf):
        dv = dbuf[pl.ds(base, 16)]
        sv = sbuf[pl.ds(base, 16)]
        m = (dv >= lo) & (dv < hi)
        cs = plsc.cumsum(m.astype(jnp.int32))
        cnt = plsc.all_reduce_population_count(m)
        packed = sv * 512 + (dv - lo)
        pos = jnp.where(m, (cposv + cs - 1) & (RING - 1), RING)
        plsc.store_scatter(cl, [pos], packed)
        return cposv + cnt

    def flush_to(state, nf_target):
        def cond(st):
            return st < nf_target

        def body(nf):
            @pl.when(nf > 0)
            def _():
                rmw_batch(nf - 1)
            start_gather(nf)
            return nf + 1
        return lax.while_loop(cond, body, state)

    start_chunk(0, 0)

    def chunk_pair(tp, state):
        cposv, nf = state
        for par in range(2):
            t = tp * 2 + par

            @pl.when(t + 1 < NCH)
            def _():
                start_chunk(t + 1, (par + 1) % 2)

            wait_chunk(par)

            def group_body(g, cp):
                return scan_group(g * 16, cp, dstb[par], srcb[par])
            cposv = lax.fori_loop(0, GPC, group_body, cposv, unroll=5)
            cpos = cposv[0]
            nf = flush_to(nf, cpos // BATCH)
        return (cposv, nf)

    cpos0 = jnp.zeros((16,), jnp.int32)
    cposv, nf = lax.fori_loop(0, NCH // 2, chunk_pair, (cpos0, 0))
    cpos = cposv[0]

    dumv = jnp.full((16,), DUMMY, jnp.int32)
    for i in range(EPB):
        pos = (cpos + i * 16 + lanes) & (RING - 1)
        plsc.store_scatter(cl, [pos], dumv)

    nf = flush_to(nf, (cpos + BATCH - 1) // BATCH)

    @pl.when(nf > 0)
    def _():
        rmw_batch(nf - 1)

    pltpu.sync_copy(acc.at[pl.ds(0, R)], out_hbm.at[pl.ds(wid * R, R)])


@functools.partial(jax.jit, static_argnames=())
def _segmax(q, src, dst):
    mesh = plsc.VectorSubcoreMesh(core_axis_name="c", subcore_axis_name="s",
                                  num_cores=NC, num_subcores=NS)
    kern = pl.kernel(
        _segmax_body,
        out_type=jax.ShapeDtypeStruct((NPAD, C), jnp.bfloat16),
        mesh=mesh,
        compiler_params=pltpu.CompilerParams(needs_layout_passes=False,
                                             use_tc_tiling_on_sc=False),
        scratch_types=[
            pltpu.VMEM((CHUNK,), jnp.int32),
            pltpu.VMEM((CHUNK,), jnp.int32),
            pltpu.VMEM((CHUNK,), jnp.int32),
            pltpu.VMEM((CHUNK,), jnp.int32),
            pltpu.VMEM((CLCAP,), jnp.int32),
            pltpu.VMEM((2 * BATCH,), jnp.int32),
            pltpu.VMEM((2 * BATCH,), jnp.int32),
            pltpu.VMEM((2 * BATCH, C // 2), jnp.int32),
            pltpu.VMEM((R + 1, C), jnp.bfloat16),
            pltpu.SemaphoreType.DMA,
            pltpu.SemaphoreType.DMA,
        ],
    )
    return kern(src, dst, q)


def _branch_edges(edge_index, d):
    ei = edge_index.reshape(2, N, KD)
    ei = ei[:, :, 0:K * d:d]
    return ei[0].reshape(-1), ei[1].reshape(-1)


def _combined_w(W, b, cin):
    wt, wb = W[:cin], W[cin:]
    wc = jnp.concatenate([wt - wb, wb], axis=1)
    bc = jnp.concatenate([b, jnp.zeros_like(b)])
    return wc, bc


def kernel(x, edge_index, W0b0, b0b0, W0b1, b0b1, W0f, b0f,
           W1b0, b1b0, W1b1, b1b1, W1f, b1f):
    params = [((W0b0, b0b0), (W0b1, b0b1), (W0f, b0f)),
              ((W1b0, b1b0), (W1b1, b1b1), (W1f, b1f))]
    per_branch = []
    for d, ((Wa, ba), (Wb, bb), _) in zip((1, 2), params):
        src, dst = _branch_edges(edge_index, d)
        wc0, bc0 = _combined_w(Wa, ba, C)
        p0, q0 = _pq_mm(x, wc0, bc0)
        q0p = lax.bitcast_convert_type(q0.reshape(N, C // 2, 2), jnp.int32)
        s0 = _segmax(q0p, src, dst)[:N]
        wc1, bc1 = _combined_w(Wb, bb, 2 * C)
        h0, p1, q1 = _h_and_mm(p0, s0, x, wc1, bc1)
        q1p = lax.bitcast_convert_type(q1.reshape(N, C // 2, 2), jnp.int32)
        s1 = _segmax(q1p, src, dst)[:N]
        per_branch.append((h0, p1, s1))
    (h00, p10, s10), (h01, p11, s11) = per_branch
    return _final(x, h00, p10, s10, h01, p11, s11,
                  params[0][2][0], params[0][2][1],
                  params[1][2][0], params[1][2][1])


# speedup vs baseline: 1.0019x; 1.0019x over previous
import functools

import jax
import jax.numpy as jnp
from jax import lax
from jax.experimental import pallas as pl
from jax.experimental.pallas import tpu as pltpu
from jax.experimental.pallas import tpu_sc as plsc

N = 10000
C = 128
K = 16
KD = 32
E = N * K
NEG = -3.0e38
THRESH = -1.0e38

NC = 2
NS = 16
NW = NC * NS
R = 320
NPAD = NW * R
DUMMY = R

CHUNK = 2000
NCH = E // CHUNK
GPC = CHUNK // 16
BATCH = 128
RCAP = 256
RING = 16 * RCAP
CLCAP = RING + 16
EPB = BATCH // 16
PRESS = RCAP - 126


def _lrelu(x):
    return jnp.where(x >= 0, x, 0.2 * x)


def _mm_kernel(x_ref, w_ref, b_ref, o_ref):
    o_ref[...] = jnp.dot(x_ref[...], w_ref[...],
                         preferred_element_type=jnp.float32) + b_ref[...]


def _mm(x, w, b):
    n, cin = x.shape
    cout = w.shape[1]
    blk = 400
    return pl.pallas_call(
        _mm_kernel,
        grid=(n // blk,),
        in_specs=[
            pl.BlockSpec((blk, cin), lambda i: (i, 0)),
            pl.BlockSpec((cin, cout), lambda i: (0, 0)),
            pl.BlockSpec((cout,), lambda i: (0,)),
        ],
        out_specs=pl.BlockSpec((blk, cout), lambda i: (i, 0)),
        out_shape=jax.ShapeDtypeStruct((n, cout), jnp.float32),
    )(x, w, b)


def _pq_kernel(x_ref, w_ref, b_ref, p_ref, q_ref):
    pq = jnp.dot(x_ref[...], w_ref[...],
                 preferred_element_type=jnp.float32) + b_ref[...]
    p_ref[...] = pq[:, :C]
    q_ref[...] = pq[:, C:].astype(jnp.bfloat16)


def _pq_mm(x, w, b):
    n, cin = x.shape
    blk = 400
    return pl.pallas_call(
        _pq_kernel,
        grid=(n // blk,),
        in_specs=[
            pl.BlockSpec((blk, cin), lambda i: (i, 0)),
            pl.BlockSpec((cin, 2 * C), lambda i: (0, 0)),
            pl.BlockSpec((2 * C,), lambda i: (0,)),
        ],
        out_specs=[
            pl.BlockSpec((blk, C), lambda i: (i, 0)),
            pl.BlockSpec((blk, C), lambda i: (i, 0)),
        ],
        out_shape=[
            jax.ShapeDtypeStruct((n, C), jnp.float32),
            jax.ShapeDtypeStruct((n, C), jnp.bfloat16),
        ],
    )(x, w, b)


def _h_mm_kernel(p_ref, s_ref, x_ref, w_ref, b_ref, h_ref, p1_ref, q1_ref):
    s = s_ref[...].astype(jnp.float32)
    h = jnp.where(s > THRESH, _lrelu(p_ref[...] + s), 0.0)
    h_ref[...] = h
    f = jnp.concatenate([x_ref[...], h], axis=-1)
    pq = jnp.dot(f, w_ref[...],
                 preferred_element_type=jnp.float32) + b_ref[...]
    p1_ref[...] = pq[:, :C]
    q1_ref[...] = pq[:, C:].astype(jnp.bfloat16)


def _h_and_mm(p, s, x, w, b):
    n = x.shape[0]
    cin = x.shape[1] + C
    blk = 400
    return pl.pallas_call(
        _h_mm_kernel,
        grid=(n // blk,),
        in_specs=[
            pl.BlockSpec((blk, C), lambda i: (i, 0)),
            pl.BlockSpec((blk, C), lambda i: (i, 0)),
            pl.BlockSpec((blk, x.shape[1]), lambda i: (i, 0)),
            pl.BlockSpec((cin, 2 * C), lambda i: (0, 0)),
            pl.BlockSpec((2 * C,), lambda i: (0,)),
        ],
        out_specs=[
            pl.BlockSpec((blk, C), lambda i: (i, 0)),
            pl.BlockSpec((blk, C), lambda i: (i, 0)),
            pl.BlockSpec((blk, C), lambda i: (i, 0)),
        ],
        out_shape=[
            jax.ShapeDtypeStruct((n, C), jnp.float32),
            jax.ShapeDtypeStruct((n, C), jnp.float32),
            jax.ShapeDtypeStruct((n, C), jnp.bfloat16),
        ],
    )(p, s, x, w, b)


def _final_kernel(x_ref, h00_ref, p10_ref, s10_ref, h01_ref, p11_ref,
                  s11_ref, wf0_ref, bf0_ref, wf1_ref, bf1_ref, o_ref):
    x = x_ref[...]
    outs = []
    for h0, p1, s1, wf, bf in (
        (h00_ref, p10_ref, s10_ref, wf0_ref, bf0_ref),
        (h01_ref, p11_ref, s11_ref, wf1_ref, bf1_ref),
    ):
        s = s1[...].astype(jnp.float32)
        h1 = jnp.where(s > THRESH, _lrelu(p1[...] + s), 0.0)
        f = jnp.concatenate([x, h0[...], h1], axis=-1)
        outs.append(jnp.dot(f, wf[...],
                            preferred_element_type=jnp.float32) + bf[...])
    o_ref[...] = jnp.maximum(outs[0], outs[1]) + x


def _final(x, h00, p10, s10, h01, p11, s11, wf0, bf0, wf1, bf1):
    n = x.shape[0]
    blk = 400
    row = lambda i: (i, 0)
    full = lambda i: (0, 0)
    return pl.pallas_call(
        _final_kernel,
        grid=(n // blk,),
        in_specs=[
            pl.BlockSpec((blk, C), row), pl.BlockSpec((blk, C), row),
            pl.BlockSpec((blk, C), row), pl.BlockSpec((blk, C), row),
            pl.BlockSpec((blk, C), row), pl.BlockSpec((blk, C), row),
            pl.BlockSpec((blk, C), row),
            pl.BlockSpec((3 * C, C), full), pl.BlockSpec((C,), lambda i: (0,)),
            pl.BlockSpec((3 * C, C), full), pl.BlockSpec((C,), lambda i: (0,)),
        ],
        out_specs=pl.BlockSpec((blk, C), row),
        out_shape=jax.ShapeDtypeStruct((n, C), jnp.float32),
    )(x, h00, p10, s10, h01, p11, s11, wf0, bf0, wf1, bf1)


def _segmax_body(src_hbm, dst_hbm, q_hbm, out_hbm,
                 dstb0, dstb1, srcb0, srcb1, cl, gidx, lst, rowb, acc,
                 sem_scan, sem_g):
    dstb = (dstb0, dstb1)
    srcb = (srcb0, srcb1)
    wid = lax.axis_index("s") * NC + lax.axis_index("c")
    lo = wid * R
    hi = lo + R
    negv = jnp.full((32,), NEG, jnp.bfloat16)

    def init_body(r, carry):
        for v in range(4):
            acc[r, pl.ds(v * 32, 32)] = negv
        return carry
    lax.fori_loop(0, R + 1, init_body, 0)

    def start_chunk(t, par):
        pltpu.make_async_copy(
            dst_hbm.at[pl.ds(t * CHUNK, CHUNK)], dstb[par], sem_scan).start()
        pltpu.make_async_copy(
            src_hbm.at[pl.ds(t * CHUNK, CHUNK)], srcb[par], sem_scan).start()

    def wait_chunk(par):
        pltpu.make_async_copy(
            dst_hbm.at[pl.ds(0, CHUNK)], dstb[par], sem_scan).wait()
        pltpu.make_async_copy(
            src_hbm.at[pl.ds(0, CHUNK)], srcb[par], sem_scan).wait()

    lanes = lax.iota(jnp.int32, 16)

    def start_gather(nf):
        par = (nf % 2) * BATCH
        fb = pl.multiple_of((nf * BATCH) % RING, BATCH)
        for i in range(EPB):
            pv = cl[pl.ds(fb + i * 16, 16)]
            gidx[pl.ds(par + i * 16, 16)] = pv >> 9
            lst[pl.ds(par + i * 16, 16)] = pv & 511
        pltpu.make_async_copy(
            q_hbm.at[gidx.at[pl.ds(par, BATCH)]],
            rowb.at[pl.ds(par, BATCH)], sem_g).start()

    def rmw_batch(nf):
        par = (nf % 2) * BATCH
        pltpu.make_async_copy(
            q_hbm.at[gidx.at[pl.ds(par, BATCH)]],
            rowb.at[pl.ds(par, BATCH)], sem_g).wait()

        def rmw_body(i, carry):
            lv = lst[pl.ds(par + i * 16, 16)]
            for j in range(16):
                ld = lv[j]
                r = par + i * 16 + j
                g = [plsc.bitcast(rowb[r, pl.ds(v * 16, 16)], jnp.bfloat16)
                     for v in range(4)]
                a = [acc[ld, pl.ds(v * 32, 32)] for v in range(4)]
                for v in range(4):
                    acc[ld, pl.ds(v * 32, 32)] = jnp.maximum(a[v], g[v])
            return carry
        lax.fori_loop(0, 8, rmw_body, 0)

    def scan_group(base, cposv, dbuf, sbuf):
        dv = dbuf[pl.ds(base, 16)]
        sv = sbuf[pl.ds(base, 16)]
        m = (dv >= lo) & (dv < hi)
        cs = plsc.cumsum(m.astype(jnp.int32))
        cnt = plsc.all_reduce_population_count(m)
        packed = sv * 512 + (dv - lo)
        pos = jnp.where(m, (cposv + cs - 1) & (RING - 1), RING)
        plsc.store_scatter(cl, [pos], packed)
        return cposv + cnt

    def flush_to(state, nf_target):
        def cond(st):
            return st < nf_target

        def body(nf):
            @pl.when(nf > 0)
            def _():
                rmw_batch(nf - 1)
            start_gather(nf)
            return nf + 1
        return lax.while_loop(cond, body, state)

    start_chunk(0, 0)

    def chunk_pair(tp, state):
        cposv, nf = state
        for par in range(2):
            t = tp * 2 + par

            @pl.when(t + 1 < NCH)
            def _():
                start_chunk(t + 1, (par + 1) % 2)

            wait_chunk(par)

            def group_body(g, cp):
                return scan_group(g * 16, cp, dstb[par], srcb[par])
            cposv = lax.fori_loop(0, GPC, group_body, cposv, unroll=5)
            cpos = cposv[0]
            nf = flush_to(nf, cpos // BATCH)
        return (cposv, nf)

    cpos0 = jnp.zeros((16,), jnp.int32)
    cposv, nf = lax.fori_loop(0, NCH // 2, chunk_pair, (cpos0, 0))
    cpos = cposv[0]

    dumv = jnp.full((16,), DUMMY, jnp.int32)
    for i in range(EPB):
        pos = (cpos + i * 16 + lanes) & (RING - 1)
        plsc.store_scatter(cl, [pos], dumv)

    nf = flush_to(nf, (cpos + BATCH - 1) // BATCH)

    @pl.when(nf > 0)
    def _():
        rmw_batch(nf - 1)

    pltpu.sync_copy(acc.at[pl.ds(0, R)], out_hbm.at[pl.ds(wid * R, R)])


@functools.partial(jax.jit, static_argnames=())
def _segmax(q, src, dst):
    mesh = plsc.VectorSubcoreMesh(core_axis_name="c", subcore_axis_name="s",
                                  num_cores=NC, num_subcores=NS)
    kern = pl.kernel(
        _segmax_body,
        out_type=jax.ShapeDtypeStruct((NPAD, C), jnp.bfloat16),
        mesh=mesh,
        compiler_params=pltpu.CompilerParams(needs_layout_passes=False,
                                             use_tc_tiling_on_sc=False),
        scratch_types=[
            pltpu.VMEM((CHUNK,), jnp.int32),
            pltpu.VMEM((CHUNK,), jnp.int32),
            pltpu.VMEM((CHUNK,), jnp.int32),
            pltpu.VMEM((CHUNK,), jnp.int32),
            pltpu.VMEM((CLCAP,), jnp.int32),
            pltpu.VMEM((2 * BATCH,), jnp.int32),
            pltpu.VMEM((2 * BATCH,), jnp.int32),
            pltpu.VMEM((2 * BATCH, C // 2), jnp.int32),
            pltpu.VMEM((R + 1, C), jnp.bfloat16),
            pltpu.SemaphoreType.DMA,
            pltpu.SemaphoreType.DMA,
        ],
    )
    return kern(src, dst, q)


def _branch_edges(edge_index, d):
    ei = edge_index.reshape(2, N, KD)
    ei = ei[:, :, 0:K * d:d]
    return ei[0].reshape(-1), ei[1].reshape(-1)


def _combined_w(W, b, cin):
    wt, wb = W[:cin], W[cin:]
    wc = jnp.concatenate([wt - wb, wb], axis=1)
    bc = jnp.concatenate([b, jnp.zeros_like(b)])
    return wc, bc


def kernel(x, edge_index, W0b0, b0b0, W0b1, b0b1, W0f, b0f,
           W1b0, b1b0, W1b1, b1b1, W1f, b1f):
    params = [((W0b0, b0b0), (W0b1, b0b1), (W0f, b0f)),
              ((W1b0, b1b0), (W1b1, b1b1), (W1f, b1f))]
    per_branch = []
    for d, ((Wa, ba), (Wb, bb), _) in zip((1, 2), params):
        src, dst = _branch_edges(edge_index, d)
        wc0, bc0 = _combined_w(Wa, ba, C)
        p0, q0 = _pq_mm(x, wc0, bc0)
        q0p = lax.bitcast_convert_type(q0.reshape(N, C // 2, 2), jnp.int32)
        s0 = _segmax(q0p, src, dst)[:N]
        wc1, bc1 = _combined_w(Wb, bb, 2 * C)
        h0, p1, q1 = _h_and_mm(p0, s0, x, wc1, bc1)
        q1p = lax.bitcast_convert_type(q1.reshape(N, C // 2, 2), jnp.int32)
        s1 = _segmax(q1p, src, dst)[:N]
        per_branch.append((h0, p1, s1))
    (h00, p10, s10), (h01, p11, s11) = per_branch
    return _final(x, h00, p10, s10, h01, p11, s11,
                  params[0][2][0], params[0][2][1],
                  params[1][2][0], params[1][2][1])
